# Initial kernel scaffold; baseline (speedup 1.0000x reference)
import jax
import jax.numpy as jnp
from jax.experimental import pallas as pl


def kernel(x, edge_index, edge_weight, graph_ids, params):
    raise NotImplementedError("write your pallas kernel here")


# speedup vs baseline: 2.1274x; 2.1274x over previous
import functools

import jax
import jax.numpy as jnp
from jax import lax
from jax.experimental import pallas as pl
from jax.experimental.pallas import tpu as pltpu
from jax.experimental.pallas import tpu_sc as plsc

N = 10000
E = 320000
D = 128
H = 512
OUT = 128
G = 32
NC = 2
NS = 16
NW = NC * NS
EB = 128
NB = 157
EPAD = NS * NB * EB
CW = 128
K0 = D // CW
K1 = H // CW
HALF = N // NC
TRASH = HALF
SLAB = 1000
NSLAB = HALF // SLAB


def _sc_agg(chunks, src3, dst3, w3, zeros_slab):
    K = len(chunks)
    mesh = plsc.VectorSubcoreMesh(core_axis_name="c", subcore_axis_name="s")
    out_types = [jax.ShapeDtypeStruct((N, CW), jnp.float32) for _ in range(K)]

    @functools.partial(
        pl.kernel,
        out_type=out_types,
        mesh=mesh,
        scratch_types=[
            pltpu.VMEM((NB, EB), jnp.int32),
            pltpu.VMEM((NB, EB), jnp.int32),
            pltpu.VMEM((NB, EB), jnp.float32),
            pltpu.VMEM((EB, CW), jnp.float32),
            pltpu.VMEM_SHARED((HALF + 8, CW), jnp.float32),
        ],
    )
    def kern(*refs):
        chunk_refs = refs[:K]
        src_hbm, dst_hbm, w_hbm, zeros_hbm = refs[K:K + 4]
        out_refs = refs[K + 4:K + 4 + K]
        src_v, dst_v, w_v, rows_v, acc = refs[K + 4 + K:]

        c = lax.axis_index("c")
        s = lax.axis_index("s")
        base = c * HALF
        row0 = pl.multiple_of(s * SLAB, 8)

        pltpu.sync_copy(src_hbm.at[s], src_v)
        pltpu.sync_copy(dst_hbm.at[s], dst_v)
        pltpu.sync_copy(w_hbm.at[s], w_v)

        @pl.loop(0, NB)
        def _(j):
            @pl.loop(0, EB, step=16)
            def _(u):
                t = dst_v[j, pl.ds(u, 16)] - base
                ok = (t >= 0) & (t < HALF)
                dst_v[j, pl.ds(u, 16)] = jnp.where(ok, t, TRASH)

        @pl.when(s < NSLAB)
        def _():
            pltpu.sync_copy(zeros_slab_ref_slice(zeros_hbm, SLAB),
                            acc.at[pl.ds(row0, SLAB)])

        @pl.when(s == NSLAB)
        def _():
            pltpu.sync_copy(zeros_slab_ref_slice(zeros_hbm, 8),
                            acc.at[pl.ds(TRASH, 8)])

        for k in range(K):
            plsc.subcore_barrier()

            @pl.loop(0, NB)
            def _(j):
                pltpu.sync_copy(chunk_refs[k].at[src_v.at[j]], rows_v)

                @pl.loop(0, EB, step=16)
                def _(r0):
                    wv = w_v[j, pl.ds(r0, 16)]
                    for rr in range(16):
                        wr = wv[rr]
                        for q in range(0, CW, 16):
                            rows_v[r0 + rr, pl.ds(q, 16)] = (
                                rows_v[r0 + rr, pl.ds(q, 16)] * wr)

                pltpu.sync_copy(rows_v, acc.at[dst_v.at[j]], add=True)

            plsc.subcore_barrier()

            @pl.when(s < NSLAB)
            def _():
                out_row = pl.multiple_of(base + row0, 8)
                pltpu.sync_copy(acc.at[pl.ds(row0, SLAB)],
                                out_refs[k].at[pl.ds(out_row, SLAB)])
                if k + 1 < K:
                    pltpu.sync_copy(zeros_slab_ref_slice(zeros_hbm, SLAB),
                                    acc.at[pl.ds(row0, SLAB)])

    return kern(*chunks, src3, dst3, w3, zeros_slab)


def zeros_slab_ref_slice(zeros_hbm, nrows):
    return zeros_hbm.at[pl.ds(0, nrows)] if nrows != SLAB else zeros_hbm


def _mlp_layer(x, p_chunks, s_eps, w1, b1, w2, b2):
    B = 1000
    grid = (N // B,)

    def body(*refs):
        x_ref = refs[0]
        q_refs = refs[1:1 + K0]
        eps_ref, w1_ref, b1_ref, w2_ref, b2_ref = refs[1 + K0:1 + K0 + 5]
        o_refs = refs[1 + K0 + 5:]
        se = eps_ref[0, 0]
        z = None
        for k in range(K0):
            pooled_k = (q_refs[k][...]
                        + se * x_ref[:, k * CW:(k + 1) * CW])
            zk = jnp.dot(pooled_k, w1_ref[pl.ds(k * CW, CW), :],
                         preferred_element_type=jnp.float32)
            z = zk if z is None else z + zk
        z = jnp.maximum(z + b1_ref[...], 0.0)
        h = jnp.dot(z, w2_ref[...], preferred_element_type=jnp.float32)
        h = jnp.maximum(h + b2_ref[...], 0.0)
        for k in range(K1):
            o_refs[k][...] = h[:, k * CW:(k + 1) * CW]

    outs = pl.pallas_call(
        body,
        grid=grid,
        in_specs=(
            [pl.BlockSpec((B, D), lambda i: (i, 0))]
            + [pl.BlockSpec((B, CW), lambda i: (i, 0))
               for _ in range(K0)]
            + [
                pl.BlockSpec(memory_space=pltpu.SMEM),
                pl.BlockSpec((D, H), lambda i: (0, 0)),
                pl.BlockSpec((1, H), lambda i: (0, 0)),
                pl.BlockSpec((H, H), lambda i: (0, 0)),
                pl.BlockSpec((1, H), lambda i: (0, 0)),
            ]
        ),
        out_specs=[pl.BlockSpec((B, CW), lambda i: (i, 0)) for _ in range(K1)],
        out_shape=[jax.ShapeDtypeStruct((N, CW), jnp.float32)
                   for _ in range(K1)],
    )(x, *p_chunks, s_eps, w1, b1, w2, b2)
    return outs


def _final_layer(x, h1c, p1, s_eps, gids3, w1, b1, w2, b2,
                 l0w, l1w, l2w, bsum):
    B = 1000
    grid = (N // B,)

    def body(*refs):
        x_ref = refs[0]
        h_refs = refs[1:1 + K1]
        q_refs = refs[1 + K1:1 + 2 * K1]
        (eps_ref, g_ref, w1_ref, b1_ref, w2_ref, b2_ref,
         l0_ref, l1_ref, l2_ref, bsum_ref, out_ref) = refs[1 + 2 * K1:]
        se = eps_ref[0, 0]

        z = None
        for k in range(K1):
            pooled_k = q_refs[k][...] + se * h_refs[k][...]
            zk = jnp.dot(pooled_k, w1_ref[pl.ds(k * CW, CW), :],
                         preferred_element_type=jnp.float32)
            z = zk if z is None else z + zk
        z = jnp.maximum(z + b1_ref[...], 0.0)
        h2 = jnp.dot(z, w2_ref[...], preferred_element_type=jnp.float32)
        h2 = jnp.maximum(h2 + b2_ref[...], 0.0)

        S = jnp.dot(x_ref[...], l0_ref[...], preferred_element_type=jnp.float32)
        for k in range(K1):
            S = S + jnp.dot(h_refs[k][...], l1_ref[pl.ds(k * CW, CW), :],
                            preferred_element_type=jnp.float32)
        S = S + jnp.dot(h2, l2_ref[...], preferred_element_type=jnp.float32)
        S = S + bsum_ref[...]

        gids = g_ref[0]
        seg = lax.broadcasted_iota(jnp.int32, (G, B), 0)
        mask = (seg == gids).astype(jnp.float32)
        part = jnp.dot(mask, S, preferred_element_type=jnp.float32)

        @pl.when(pl.program_id(0) == 0)
        def _():
            out_ref[...] = jnp.zeros_like(out_ref)

        out_ref[...] += part

    return pl.pallas_call(
        body,
        grid=grid,
        in_specs=(
            [pl.BlockSpec((B, D), lambda i: (i, 0))]
            + [pl.BlockSpec((B, CW), lambda i: (i, 0)) for _ in range(K1)]
            + [pl.BlockSpec((B, CW), lambda i: (i, 0)) for _ in range(K1)]
            + [
                pl.BlockSpec(memory_space=pltpu.SMEM),
                pl.BlockSpec((1, 1, B), lambda i: (i, 0, 0)),
                pl.BlockSpec((H, H), lambda i: (0, 0)),
                pl.BlockSpec((1, H), lambda i: (0, 0)),
                pl.BlockSpec((H, H), lambda i: (0, 0)),
                pl.BlockSpec((1, H), lambda i: (0, 0)),
                pl.BlockSpec((D, OUT), lambda i: (0, 0)),
                pl.BlockSpec((H, OUT), lambda i: (0, 0)),
                pl.BlockSpec((H, OUT), lambda i: (0, 0)),
                pl.BlockSpec((1, OUT), lambda i: (0, 0)),
            ]
        ),
        out_specs=pl.BlockSpec((G, OUT), lambda i: (0, 0)),
        out_shape=jax.ShapeDtypeStruct((G, OUT), jnp.float32),
    )(x, *h1c, *p1, s_eps, gids3, w1, b1, w2, b2, l0w, l1w, l2w, bsum)


def _fold_bn(dense, bn):
    a = bn["gamma"] / jnp.sqrt(bn["var"] + 1e-3)
    c = bn["beta"] - bn["mean"] * a
    return dense["W"] * a[None, :], (dense["b"] * a + c)[None, :]


def kernel(x, edge_index, edge_weight, graph_ids, params):
    pad = EPAD - E
    ipad = jnp.zeros((pad,), jnp.int32)
    src3 = jnp.concatenate([edge_index[0], ipad]).reshape(NS, NB, EB)
    dst3 = jnp.concatenate([edge_index[1], ipad]).reshape(NS, NB, EB)
    w3 = jnp.concatenate([edge_weight,
                          jnp.zeros((pad,), jnp.float32)]).reshape(NS, NB, EB)
    zeros_slab = jnp.zeros((SLAB, CW), jnp.float32)
    gids3 = graph_ids.reshape(N // 1000, 1, 1000)

    lp0, lp1 = params["layers"][0], params["layers"][1]
    w1a, b1a = _fold_bn(lp0["d1"], lp0["bn1"])
    w2a, b2a = _fold_bn(lp0["d2"], lp0["bn2"])
    w1b, b1b = _fold_bn(lp1["d1"], lp1["bn1"])
    w2b, b2b = _fold_bn(lp1["d2"], lp1["bn2"])
    lin = params["linears"]
    bsum = (lin[0]["b"] + lin[1]["b"] + lin[2]["b"])[None, :]
    s0 = (1.0 + params["eps"][0]).reshape(1, 1)
    s1 = (1.0 + params["eps"][1]).reshape(1, 1)

    x_chunks = [x[:, k * CW:(k + 1) * CW] for k in range(K0)] if K0 > 1 else [x]
    p0 = _sc_agg(x_chunks, src3, dst3, w3, zeros_slab)
    h1c = _mlp_layer(x, p0, s0, w1a, b1a, w2a, b2a)
    p1 = _sc_agg(list(h1c), src3, dst3, w3, zeros_slab)
    return _final_layer(x, list(h1c), list(p1), s1, gids3, w1b, b1b, w2b, b2b,
                        lin[0]["W"], lin[1]["W"], lin[2]["W"], bsum)


# speedup vs baseline: 2.3088x; 1.0852x over previous
import dataclasses
import functools

import jax
import jax.numpy as jnp
from jax import lax
from jax.experimental import pallas as pl
from jax.experimental.pallas import tpu as pltpu
from jax.experimental.pallas import tpu_sc as plsc

N = 10000
E = 320000
D = 128
H = 512
OUT = 128
G = 32
NC = 2
NS = 16
NW = NC * NS
EB = 128
NB = 80
EPAD = NW * NB * EB
CW = 128
K0 = D // CW
K1 = H // CW
SLAB = 1000
NSLAB = N // SLAB


def _sc_agg(chunks, epk, zeros_slab):
    K = len(chunks)
    mesh = plsc.VectorSubcoreMesh(core_axis_name="c", subcore_axis_name="s")
    out_types = [jax.ShapeDtypeStruct((NC, N, CW), jnp.float32) for _ in range(K)]
    cp = pltpu.CompilerParams()
    if "needs_layout_passes" in pltpu.CompilerParams.__dataclass_fields__:
        cp = dataclasses.replace(cp, needs_layout_passes=False)

    @functools.partial(
        pl.kernel,
        out_type=out_types,
        mesh=mesh,
        compiler_params=cp,
        scratch_types=(
            [pltpu.VMEM((3, EB), jnp.int32) for _ in range(4)]
            + [pltpu.VMEM((EB, CW), jnp.float32) for _ in range(2)]
            + [pltpu.VMEM_SHARED((N, CW), jnp.float32)]
            + [pltpu.SemaphoreType.DMA for _ in range(6)]
        ),
    )
    def kern(*refs):
        chunk_refs = refs[:K]
        epk_hbm, zeros_hbm = refs[K:K + 2]
        out_refs = refs[K + 2:K + 2 + K]
        sc = refs[K + 2 + K:]
        ebuf = sc[0:4]
        rbuf = sc[4:6]
        acc = sc[6]
        esem = sc[7:11]
        rsem = sc[11:13]

        c = lax.axis_index("c")
        s = lax.axis_index("s")
        wid = c * NS + s
        row0 = pl.multiple_of(s * SLAB, 8)

        def ecopy(b, t):
            return pltpu.make_async_copy(epk_hbm.at[wid, b], ebuf[t], esem[t])

        def scale_rows(t):
            @pl.loop(0, EB, step=16)
            def _(r0):
                wv = plsc.bitcast(ebuf[t][2, pl.ds(r0, 16)], jnp.float32)
                for rr in range(16):
                    wr = wv[rr]
                    for q in range(0, CW, 16):
                        rbuf[t % 2][r0 + rr, pl.ds(q, 16)] = (
                            rbuf[t % 2][r0 + rr, pl.ds(q, 16)] * wr)

        @pl.when(s < NSLAB)
        def _():
            pltpu.sync_copy(zeros_hbm, acc.at[pl.ds(row0, SLAB)])

        for k in range(K):
            plsc.subcore_barrier()

            def gcopy(t, rt):
                return pltpu.make_async_copy(
                    chunk_refs[k].at[ebuf[t].at[0]], rbuf[rt], rsem[rt])

            for t in range(3):
                ecopy(t, t).start()
            ecopy(0, 0).wait()
            gcopy(0, 0).start()
            ecopy(1, 1).wait()
            gcopy(1, 1).start()

            @pl.loop(0, NB, step=4)
            def _(j):
                for t in range(4):
                    b = j + t
                    gcopy(t, t % 2).wait()
                    @pl.when(b + 3 < NB)
                    def _():
                        ecopy(b + 3, (t + 3) % 4).start()
                    scale_rows(t)
                    pltpu.sync_copy(rbuf[t % 2], acc.at[ebuf[t].at[1]],
                                    add=True)
                    @pl.when(b + 2 < NB)
                    def _():
                        ecopy(b + 2, (t + 2) % 4).wait()
                        gcopy((t + 2) % 4, t % 2).start()

            plsc.subcore_barrier()

            @pl.when(s < NSLAB)
            def _():
                pltpu.sync_copy(acc.at[pl.ds(row0, SLAB)],
                                out_refs[k].at[c, pl.ds(row0, SLAB)])
                if k + 1 < K:
                    pltpu.sync_copy(zeros_hbm, acc.at[pl.ds(row0, SLAB)])

    return kern(*chunks, epk, zeros_slab)


def _mlp_layer(x, p_chunks, s_eps, w1, b1, w2, b2):
    B = 1000
    grid = (N // B,)

    def body(*refs):
        x_ref = refs[0]
        q_refs = refs[1:1 + K0]
        eps_ref, w1_ref, b1_ref, w2_ref, b2_ref = refs[1 + K0:1 + K0 + 5]
        o_refs = refs[1 + K0 + 5:]
        se = eps_ref[0, 0]
        z = None
        for k in range(K0):
            pooled_k = (q_refs[k][0] + q_refs[k][1]
                        + se * x_ref[:, k * CW:(k + 1) * CW])
            zk = jnp.dot(pooled_k, w1_ref[pl.ds(k * CW, CW), :],
                         preferred_element_type=jnp.float32)
            z = zk if z is None else z + zk
        z = jnp.maximum(z + b1_ref[...], 0.0)
        h = jnp.dot(z, w2_ref[...], preferred_element_type=jnp.float32)
        h = jnp.maximum(h + b2_ref[...], 0.0)
        for k in range(K1):
            o_refs[k][...] = h[:, k * CW:(k + 1) * CW]

    outs = pl.pallas_call(
        body,
        grid=grid,
        in_specs=(
            [pl.BlockSpec((B, D), lambda i: (i, 0))]
            + [pl.BlockSpec((NC, B, CW), lambda i: (0, i, 0))
               for _ in range(K0)]
            + [
                pl.BlockSpec(memory_space=pltpu.SMEM),
                pl.BlockSpec((D, H), lambda i: (0, 0)),
                pl.BlockSpec((1, H), lambda i: (0, 0)),
                pl.BlockSpec((H, H), lambda i: (0, 0)),
                pl.BlockSpec((1, H), lambda i: (0, 0)),
            ]
        ),
        out_specs=[pl.BlockSpec((B, CW), lambda i: (i, 0)) for _ in range(K1)],
        out_shape=[jax.ShapeDtypeStruct((N, CW), jnp.float32)
                   for _ in range(K1)],
    )(x, *p_chunks, s_eps, w1, b1, w2, b2)
    return outs


def _final_layer(x, h1c, p1, s_eps, gids3, w1, b1, w2, b2,
                 l0w, l1w, l2w, bsum):
    B = 1000
    grid = (N // B,)

    def body(*refs):
        x_ref = refs[0]
        h_refs = refs[1:1 + K1]
        q_refs = refs[1 + K1:1 + 2 * K1]
        (eps_ref, g_ref, w1_ref, b1_ref, w2_ref, b2_ref,
         l0_ref, l1_ref, l2_ref, bsum_ref, out_ref) = refs[1 + 2 * K1:]
        se = eps_ref[0, 0]

        z = None
        for k in range(K1):
            pooled_k = q_refs[k][0] + q_refs[k][1] + se * h_refs[k][...]
            zk = jnp.dot(pooled_k, w1_ref[pl.ds(k * CW, CW), :],
                         preferred_element_type=jnp.float32)
            z = zk if z is None else z + zk
        z = jnp.maximum(z + b1_ref[...], 0.0)
        h2 = jnp.dot(z, w2_ref[...], preferred_element_type=jnp.float32)
        h2 = jnp.maximum(h2 + b2_ref[...], 0.0)

        S = jnp.dot(x_ref[...], l0_ref[...], preferred_element_type=jnp.float32)
        for k in range(K1):
            S = S + jnp.dot(h_refs[k][...], l1_ref[pl.ds(k * CW, CW), :],
                            preferred_element_type=jnp.float32)
        S = S + jnp.dot(h2, l2_ref[...], preferred_element_type=jnp.float32)
        S = S + bsum_ref[...]

        gids = g_ref[0]
        seg = lax.broadcasted_iota(jnp.int32, (G, B), 0)
        mask = (seg == gids).astype(jnp.float32)
        part = jnp.dot(mask, S, preferred_element_type=jnp.float32)

        @pl.when(pl.program_id(0) == 0)
        def _():
            out_ref[...] = jnp.zeros_like(out_ref)

        out_ref[...] += part

    return pl.pallas_call(
        body,
        grid=grid,
        in_specs=(
            [pl.BlockSpec((B, D), lambda i: (i, 0))]
            + [pl.BlockSpec((B, CW), lambda i: (i, 0)) for _ in range(K1)]
            + [pl.BlockSpec((NC, B, CW), lambda i: (0, i, 0))
               for _ in range(K1)]
            + [
                pl.BlockSpec(memory_space=pltpu.SMEM),
                pl.BlockSpec((1, 1, B), lambda i: (i, 0, 0)),
                pl.BlockSpec((H, H), lambda i: (0, 0)),
                pl.BlockSpec((1, H), lambda i: (0, 0)),
                pl.BlockSpec((H, H), lambda i: (0, 0)),
                pl.BlockSpec((1, H), lambda i: (0, 0)),
                pl.BlockSpec((D, OUT), lambda i: (0, 0)),
                pl.BlockSpec((H, OUT), lambda i: (0, 0)),
                pl.BlockSpec((H, OUT), lambda i: (0, 0)),
                pl.BlockSpec((1, OUT), lambda i: (0, 0)),
            ]
        ),
        out_specs=pl.BlockSpec((G, OUT), lambda i: (0, 0)),
        out_shape=jax.ShapeDtypeStruct((G, OUT), jnp.float32),
    )(x, *h1c, *p1, s_eps, gids3, w1, b1, w2, b2, l0w, l1w, l2w, bsum)


def _fold_bn(dense, bn):
    a = bn["gamma"] / jnp.sqrt(bn["var"] + 1e-3)
    c = bn["beta"] - bn["mean"] * a
    return dense["W"] * a[None, :], (dense["b"] * a + c)[None, :]


def kernel(x, edge_index, edge_weight, graph_ids, params):
    pad = EPAD - E
    ipad = jnp.zeros((pad,), jnp.int32)
    src_r = jnp.concatenate([edge_index[0], ipad]).reshape(NW, NB, EB)
    dst_r = jnp.concatenate([edge_index[1], ipad]).reshape(NW, NB, EB)
    wbits = jax.lax.bitcast_convert_type(
        jnp.concatenate([edge_weight, jnp.zeros((pad,), jnp.float32)]),
        jnp.int32).reshape(NW, NB, EB)
    epk = jnp.stack([src_r, dst_r, wbits], axis=2)
    zeros_slab = jnp.zeros((SLAB, CW), jnp.float32)
    gids3 = graph_ids.reshape(N // 1000, 1, 1000)

    lp0, lp1 = params["layers"][0], params["layers"][1]
    w1a, b1a = _fold_bn(lp0["d1"], lp0["bn1"])
    w2a, b2a = _fold_bn(lp0["d2"], lp0["bn2"])
    w1b, b1b = _fold_bn(lp1["d1"], lp1["bn1"])
    w2b, b2b = _fold_bn(lp1["d2"], lp1["bn2"])
    lin = params["linears"]
    bsum = (lin[0]["b"] + lin[1]["b"] + lin[2]["b"])[None, :]
    s0 = (1.0 + params["eps"][0]).reshape(1, 1)
    s1 = (1.0 + params["eps"][1]).reshape(1, 1)

    x_chunks = [x[:, k * CW:(k + 1) * CW] for k in range(K0)] if K0 > 1 else [x]
    p0 = _sc_agg(x_chunks, epk, zeros_slab)
    h1c = _mlp_layer(x, p0, s0, w1a, b1a, w2a, b2a)
    p1 = _sc_agg(list(h1c), epk, zeros_slab)
    return _final_layer(x, list(h1c), list(p1), s1, gids3, w1b, b1b, w2b, b2b,
                        lin[0]["W"], lin[1]["W"], lin[2]["W"], bsum)


# speedup vs baseline: 7.4452x; 3.2248x over previous
import dataclasses
import functools

import jax
import jax.numpy as jnp
from jax import lax
from jax.experimental import pallas as pl
from jax.experimental.pallas import tpu as pltpu
from jax.experimental.pallas import tpu_sc as plsc

N = 10000
E = 320000
D = 128
H = 512
OUT = 128
G = 32
NC = 2
NS = 16
NW = NC * NS
EB = 128
NB = 80
EPAD = NW * NB * EB
CW = 128
K0 = D // CW
K1 = H // CW
SLAB = 1000
NSLAB = N // SLAB


def _sc_agg(chunks, epk, zeros_slab):
    K = len(chunks)
    mesh = plsc.VectorSubcoreMesh(core_axis_name="c", subcore_axis_name="s")
    out_types = [jax.ShapeDtypeStruct((NC, N, CW), jnp.float32) for _ in range(K)]
    cp = pltpu.CompilerParams()
    if "needs_layout_passes" in pltpu.CompilerParams.__dataclass_fields__:
        cp = dataclasses.replace(cp, needs_layout_passes=False)

    @functools.partial(
        pl.kernel,
        out_type=out_types,
        mesh=mesh,
        compiler_params=cp,
        scratch_types=(
            [pltpu.VMEM((3, EB), jnp.int32) for _ in range(4)]
            + [pltpu.VMEM((EB, CW), jnp.float32) for _ in range(2)]
            + [pltpu.VMEM_SHARED((N, CW), jnp.float32)]
            + [pltpu.SemaphoreType.DMA for _ in range(6)]
        ),
    )
    def kern(*refs):
        chunk_refs = refs[:K]
        epk_hbm, zeros_hbm = refs[K:K + 2]
        out_refs = refs[K + 2:K + 2 + K]
        sc = refs[K + 2 + K:]
        ebuf = sc[0:4]
        rbuf = sc[4:6]
        acc = sc[6]
        esem = sc[7:11]
        rsem = sc[11:13]

        c = lax.axis_index("c")
        s = lax.axis_index("s")
        wid = c * NS + s
        row0 = pl.multiple_of(s * SLAB, 8)

        def ecopy(b, t):
            return pltpu.make_async_copy(epk_hbm.at[wid, b], ebuf[t], esem[t])

        def scale_rows(t):
            @pl.loop(0, EB, step=16)
            def _(r0):
                wv = plsc.bitcast(ebuf[t][2, pl.ds(r0, 16)], jnp.float32)
                for rr in range(16):
                    wr = wv[rr]
                    for q in range(0, CW, 16):
                        rbuf[t % 2][r0 + rr, pl.ds(q, 16)] = (
                            rbuf[t % 2][r0 + rr, pl.ds(q, 16)] * wr)

        @pl.when(s < NSLAB)
        def _():
            pltpu.sync_copy(zeros_hbm, acc.at[pl.ds(row0, SLAB)])

        for k in range(K):
            plsc.subcore_barrier()

            def gcopy(t, rt):
                return pltpu.make_async_copy(
                    chunk_refs[k].at[ebuf[t].at[0]], rbuf[rt], rsem[rt])

            for t in range(3):
                ecopy(t, t).start()
            ecopy(0, 0).wait()
            gcopy(0, 0).start()
            ecopy(1, 1).wait()
            gcopy(1, 1).start()

            @pl.loop(0, NB, step=4)
            def _(j):
                for t in range(4):
                    b = j + t
                    gcopy(t, t % 2).wait()
                    @pl.when(b + 3 < NB)
                    def _():
                        ecopy(b + 3, (t + 3) % 4).start()
                    scale_rows(t)
                    pltpu.sync_copy(rbuf[t % 2], acc.at[ebuf[t].at[1]],
                                    add=True)
                    @pl.when(b + 2 < NB)
                    def _():
                        ecopy(b + 2, (t + 2) % 4).wait()
                        gcopy((t + 2) % 4, t % 2).start()

            plsc.subcore_barrier()

            @pl.when(s < NSLAB)
            def _():
                pltpu.sync_copy(acc.at[pl.ds(row0, SLAB)],
                                out_refs[k].at[c, pl.ds(row0, SLAB)])
                if k + 1 < K:
                    pltpu.sync_copy(zeros_hbm, acc.at[pl.ds(row0, SLAB)])

    return kern(*chunks, epk, zeros_slab)


def _mlp_layer(x, p_chunks, s_eps, w1, b1, w2, b2):
    B = 1000
    grid = (N // B,)

    def body(*refs):
        x_ref = refs[0]
        q_refs = refs[1:1 + K0]
        eps_ref, w1_ref, b1_ref, w2_ref, b2_ref = refs[1 + K0:1 + K0 + 5]
        o_refs = refs[1 + K0 + 5:]
        se = eps_ref[0, 0]
        z = None
        for k in range(K0):
            pooled_k = (q_refs[k][0] + q_refs[k][1]
                        + se * x_ref[:, k * CW:(k + 1) * CW])
            zk = jnp.dot(pooled_k, w1_ref[pl.ds(k * CW, CW), :],
                         preferred_element_type=jnp.float32)
            z = zk if z is None else z + zk
        z = jnp.maximum(z + b1_ref[...], 0.0)
        h = jnp.dot(z, w2_ref[...], preferred_element_type=jnp.float32)
        h = jnp.maximum(h + b2_ref[...], 0.0)
        for k in range(K1):
            o_refs[k][...] = h[:, k * CW:(k + 1) * CW]

    outs = pl.pallas_call(
        body,
        grid=grid,
        in_specs=(
            [pl.BlockSpec((B, D), lambda i: (i, 0))]
            + [pl.BlockSpec((NC, B, CW), lambda i: (0, i, 0))
               for _ in range(K0)]
            + [
                pl.BlockSpec(memory_space=pltpu.SMEM),
                pl.BlockSpec((D, H), lambda i: (0, 0)),
                pl.BlockSpec((1, H), lambda i: (0, 0)),
                pl.BlockSpec((H, H), lambda i: (0, 0)),
                pl.BlockSpec((1, H), lambda i: (0, 0)),
            ]
        ),
        out_specs=[pl.BlockSpec((B, CW), lambda i: (i, 0)) for _ in range(K1)],
        out_shape=[jax.ShapeDtypeStruct((N, CW), jnp.float32)
                   for _ in range(K1)],
    )(x, *p_chunks, s_eps, w1, b1, w2, b2)
    return outs


def _final_layer(x, h1c, p1, s_eps, gids3, w1, b1, w2, b2,
                 l0w, l1w, l2w, bsum):
    B = 1000
    grid = (N // B,)

    def body(*refs):
        x_ref = refs[0]
        h_refs = refs[1:1 + K1]
        q_refs = refs[1 + K1:1 + 2 * K1]
        (eps_ref, g_ref, w1_ref, b1_ref, w2_ref, b2_ref,
         l0_ref, l1_ref, l2_ref, bsum_ref, out_ref) = refs[1 + 2 * K1:]
        se = eps_ref[0, 0]

        z = None
        for k in range(K1):
            pooled_k = q_refs[k][0] + q_refs[k][1] + se * h_refs[k][...]
            zk = jnp.dot(pooled_k, w1_ref[pl.ds(k * CW, CW), :],
                         preferred_element_type=jnp.float32)
            z = zk if z is None else z + zk
        z = jnp.maximum(z + b1_ref[...], 0.0)
        h2 = jnp.dot(z, w2_ref[...], preferred_element_type=jnp.float32)
        h2 = jnp.maximum(h2 + b2_ref[...], 0.0)

        S = jnp.dot(x_ref[...], l0_ref[...], preferred_element_type=jnp.float32)
        for k in range(K1):
            S = S + jnp.dot(h_refs[k][...], l1_ref[pl.ds(k * CW, CW), :],
                            preferred_element_type=jnp.float32)
        S = S + jnp.dot(h2, l2_ref[...], preferred_element_type=jnp.float32)
        S = S + bsum_ref[...]

        gids = g_ref[0]
        seg = lax.broadcasted_iota(jnp.int32, (G, B), 0)
        mask = (seg == gids).astype(jnp.float32)
        part = jnp.dot(mask, S, preferred_element_type=jnp.float32)

        @pl.when(pl.program_id(0) == 0)
        def _():
            out_ref[...] = jnp.zeros_like(out_ref)

        out_ref[...] += part

    return pl.pallas_call(
        body,
        grid=grid,
        in_specs=(
            [pl.BlockSpec((B, D), lambda i: (i, 0))]
            + [pl.BlockSpec((B, CW), lambda i: (i, 0)) for _ in range(K1)]
            + [pl.BlockSpec((NC, B, CW), lambda i: (0, i, 0))
               for _ in range(K1)]
            + [
                pl.BlockSpec(memory_space=pltpu.SMEM),
                pl.BlockSpec((1, 1, B), lambda i: (i, 0, 0)),
                pl.BlockSpec((H, H), lambda i: (0, 0)),
                pl.BlockSpec((1, H), lambda i: (0, 0)),
                pl.BlockSpec((H, H), lambda i: (0, 0)),
                pl.BlockSpec((1, H), lambda i: (0, 0)),
                pl.BlockSpec((D, OUT), lambda i: (0, 0)),
                pl.BlockSpec((H, OUT), lambda i: (0, 0)),
                pl.BlockSpec((H, OUT), lambda i: (0, 0)),
                pl.BlockSpec((1, OUT), lambda i: (0, 0)),
            ]
        ),
        out_specs=pl.BlockSpec((G, OUT), lambda i: (0, 0)),
        out_shape=jax.ShapeDtypeStruct((G, OUT), jnp.float32),
    )(x, *h1c, *p1, s_eps, gids3, w1, b1, w2, b2, l0w, l1w, l2w, bsum)


def _fold_bn(dense, bn):
    a = bn["gamma"] / jnp.sqrt(bn["var"] + 1e-3)
    c = bn["beta"] - bn["mean"] * a
    return dense["W"] * a[None, :], (dense["b"] * a + c)[None, :]


def kernel(x, edge_index, edge_weight, graph_ids, params):
    pad = EPAD - E
    ipad = jnp.arange(pad, dtype=jnp.int32) % N
    src_r = jnp.concatenate([edge_index[0], ipad]).reshape(NW, NB, EB)
    dst_r = jnp.concatenate([edge_index[1], ipad]).reshape(NW, NB, EB)
    wbits = jax.lax.bitcast_convert_type(
        jnp.concatenate([edge_weight, jnp.zeros((pad,), jnp.float32)]),
        jnp.int32).reshape(NW, NB, EB)
    epk = jnp.stack([src_r, dst_r, wbits], axis=2)
    zeros_slab = jnp.zeros((SLAB, CW), jnp.float32)
    gids3 = graph_ids.reshape(N // 1000, 1, 1000)

    lp0, lp1 = params["layers"][0], params["layers"][1]
    w1a, b1a = _fold_bn(lp0["d1"], lp0["bn1"])
    w2a, b2a = _fold_bn(lp0["d2"], lp0["bn2"])
    w1b, b1b = _fold_bn(lp1["d1"], lp1["bn1"])
    w2b, b2b = _fold_bn(lp1["d2"], lp1["bn2"])
    lin = params["linears"]
    bsum = (lin[0]["b"] + lin[1]["b"] + lin[2]["b"])[None, :]
    s0 = (1.0 + params["eps"][0]).reshape(1, 1)
    s1 = (1.0 + params["eps"][1]).reshape(1, 1)

    x_chunks = [x[:, k * CW:(k + 1) * CW] for k in range(K0)] if K0 > 1 else [x]
    p0 = _sc_agg(x_chunks, epk, zeros_slab)
    h1c = _mlp_layer(x, p0, s0, w1a, b1a, w2a, b2a)
    p1 = _sc_agg(list(h1c), epk, zeros_slab)
    return _final_layer(x, list(h1c), list(p1), s1, gids3, w1b, b1b, w2b, b2b,
                        lin[0]["W"], lin[1]["W"], lin[2]["W"], bsum)


# speedup vs baseline: 7.7487x; 1.0408x over previous
import dataclasses
import functools

import jax
import jax.numpy as jnp
from jax import lax
from jax.experimental import pallas as pl
from jax.experimental.pallas import tpu as pltpu
from jax.experimental.pallas import tpu_sc as plsc

N = 10000
E = 320000
D = 128
H = 512
OUT = 128
G = 32
NC = 2
NS = 16
NW = NC * NS
EB = 80
NB = 128
EPAD = NW * NB * EB
CW = 128
K0 = D // CW
K1 = H // CW
SLAB = 1000
NSLAB = N // SLAB


def _sc_agg(chunks, epk, zeros_slab):
    K = len(chunks)
    mesh = plsc.VectorSubcoreMesh(core_axis_name="c", subcore_axis_name="s")
    out_types = [jax.ShapeDtypeStruct((NC, N, CW), jnp.float32) for _ in range(K)]
    cp = pltpu.CompilerParams()
    if "needs_layout_passes" in pltpu.CompilerParams.__dataclass_fields__:
        cp = dataclasses.replace(cp, needs_layout_passes=False)

    @functools.partial(
        pl.kernel,
        out_type=out_types,
        mesh=mesh,
        compiler_params=cp,
        scratch_types=(
            [pltpu.VMEM((3, EB), jnp.int32) for _ in range(4)]
            + [pltpu.VMEM((EB, CW), jnp.float32) for _ in range(4)]
            + [pltpu.VMEM((4, EB), jnp.int32)]
            + [pltpu.VMEM_SHARED((N, CW), jnp.float32)]
            + [pltpu.SemaphoreType.DMA for _ in range(12)]
        ),
    )
    def kern(*refs):
        chunk_refs = refs[:K]
        epk_hbm, zeros_hbm = refs[K:K + 2]
        out_refs = refs[K + 2:K + 2 + K]
        sc = refs[K + 2 + K:]
        ebuf = sc[0:4]
        rbuf = sc[4:8]
        dstb = sc[8]
        acc = sc[9]
        esem = sc[10:14]
        rsem = sc[14:18]
        ssem = sc[18:22]

        c = lax.axis_index("c")
        s = lax.axis_index("s")
        wid = c * NS + s
        row0 = pl.multiple_of(s * SLAB, 8)

        def ecopy(b, t):
            return pltpu.make_async_copy(epk_hbm.at[wid, b], ebuf[t], esem[t])

        def swait(t):
            pltpu.make_async_copy(rbuf[t], acc.at[dstb.at[t]], ssem[t]).wait()

        def scale_rows(t):
            @pl.loop(0, EB, step=16)
            def _(r0):
                wv = plsc.bitcast(ebuf[t][2, pl.ds(r0, 16)], jnp.float32)
                for rr in range(16):
                    wr = wv[rr]
                    for q in range(0, CW, 16):
                        rbuf[t][r0 + rr, pl.ds(q, 16)] = (
                            rbuf[t][r0 + rr, pl.ds(q, 16)] * wr)

        @pl.when(s < NSLAB)
        def _():
            pltpu.sync_copy(zeros_hbm, acc.at[pl.ds(row0, SLAB)])

        for k in range(K):
            plsc.subcore_barrier()

            def gcopy(t):
                return pltpu.make_async_copy(
                    chunk_refs[k].at[ebuf[t].at[0]], rbuf[t], rsem[t])

            for t in range(3):
                ecopy(t, t).start()
            ecopy(0, 0).wait()
            gcopy(0).start()
            ecopy(1, 1).wait()
            gcopy(1).start()

            @pl.loop(0, NB, step=4)
            def _(j):
                for t in range(4):
                    b = j + t
                    gcopy(t).wait()
                    @pl.when(b + 3 < NB)
                    def _():
                        ecopy(b + 3, (t + 3) % 4).start()
                    scale_rows(t)
                    for q in range(0, EB, 16):
                        dstb[t, pl.ds(q, 16)] = ebuf[t][1, pl.ds(q, 16)]
                    pltpu.async_copy(rbuf[t], acc.at[dstb.at[t]], ssem[t],
                                     add=True)
                    @pl.when(b + 2 < NB)
                    def _():
                        ecopy(b + 2, (t + 2) % 4).wait()
                        @pl.when(b >= 2)
                        def _():
                            swait((t + 2) % 4)
                        gcopy((t + 2) % 4).start()

            for t in range(4):
                swait(t)

            plsc.subcore_barrier()

            @pl.when(s < NSLAB)
            def _():
                pltpu.sync_copy(acc.at[pl.ds(row0, SLAB)],
                                out_refs[k].at[c, pl.ds(row0, SLAB)])
                if k + 1 < K:
                    pltpu.sync_copy(zeros_hbm, acc.at[pl.ds(row0, SLAB)])

    return kern(*chunks, epk, zeros_slab)


def _mlp_layer(x, p_chunks, s_eps, w1, b1, w2, b2):
    B = 1000
    grid = (N // B,)

    def body(*refs):
        x_ref = refs[0]
        q_refs = refs[1:1 + K0]
        eps_ref, w1_ref, b1_ref, w2_ref, b2_ref = refs[1 + K0:1 + K0 + 5]
        o_refs = refs[1 + K0 + 5:]
        se = eps_ref[0, 0]
        z = None
        for k in range(K0):
            pooled_k = (q_refs[k][0] + q_refs[k][1]
                        + se * x_ref[:, k * CW:(k + 1) * CW])
            zk = jnp.dot(pooled_k, w1_ref[pl.ds(k * CW, CW), :],
                         preferred_element_type=jnp.float32)
            z = zk if z is None else z + zk
        z = jnp.maximum(z + b1_ref[...], 0.0)
        h = jnp.dot(z, w2_ref[...], preferred_element_type=jnp.float32)
        h = jnp.maximum(h + b2_ref[...], 0.0)
        for k in range(K1):
            o_refs[k][...] = h[:, k * CW:(k + 1) * CW]

    outs = pl.pallas_call(
        body,
        grid=grid,
        in_specs=(
            [pl.BlockSpec((B, D), lambda i: (i, 0))]
            + [pl.BlockSpec((NC, B, CW), lambda i: (0, i, 0))
               for _ in range(K0)]
            + [
                pl.BlockSpec(memory_space=pltpu.SMEM),
                pl.BlockSpec((D, H), lambda i: (0, 0)),
                pl.BlockSpec((1, H), lambda i: (0, 0)),
                pl.BlockSpec((H, H), lambda i: (0, 0)),
                pl.BlockSpec((1, H), lambda i: (0, 0)),
            ]
        ),
        out_specs=[pl.BlockSpec((B, CW), lambda i: (i, 0)) for _ in range(K1)],
        out_shape=[jax.ShapeDtypeStruct((N, CW), jnp.float32)
                   for _ in range(K1)],
    )(x, *p_chunks, s_eps, w1, b1, w2, b2)
    return outs


def _final_layer(x, h1c, p1, s_eps, gids3, w1, b1, w2, b2,
                 l0w, l1w, l2w, bsum):
    B = 1000
    grid = (N // B,)

    def body(*refs):
        x_ref = refs[0]
        h_refs = refs[1:1 + K1]
        q_refs = refs[1 + K1:1 + 2 * K1]
        (eps_ref, g_ref, w1_ref, b1_ref, w2_ref, b2_ref,
         l0_ref, l1_ref, l2_ref, bsum_ref, out_ref) = refs[1 + 2 * K1:]
        se = eps_ref[0, 0]

        z = None
        for k in range(K1):
            pooled_k = q_refs[k][0] + q_refs[k][1] + se * h_refs[k][...]
            zk = jnp.dot(pooled_k, w1_ref[pl.ds(k * CW, CW), :],
                         preferred_element_type=jnp.float32)
            z = zk if z is None else z + zk
        z = jnp.maximum(z + b1_ref[...], 0.0)
        h2 = jnp.dot(z, w2_ref[...], preferred_element_type=jnp.float32)
        h2 = jnp.maximum(h2 + b2_ref[...], 0.0)

        S = jnp.dot(x_ref[...], l0_ref[...], preferred_element_type=jnp.float32)
        for k in range(K1):
            S = S + jnp.dot(h_refs[k][...], l1_ref[pl.ds(k * CW, CW), :],
                            preferred_element_type=jnp.float32)
        S = S + jnp.dot(h2, l2_ref[...], preferred_element_type=jnp.float32)
        S = S + bsum_ref[...]

        gids = g_ref[0]
        seg = lax.broadcasted_iota(jnp.int32, (G, B), 0)
        mask = (seg == gids).astype(jnp.float32)
        part = jnp.dot(mask, S, preferred_element_type=jnp.float32)

        @pl.when(pl.program_id(0) == 0)
        def _():
            out_ref[...] = jnp.zeros_like(out_ref)

        out_ref[...] += part

    return pl.pallas_call(
        body,
        grid=grid,
        in_specs=(
            [pl.BlockSpec((B, D), lambda i: (i, 0))]
            + [pl.BlockSpec((B, CW), lambda i: (i, 0)) for _ in range(K1)]
            + [pl.BlockSpec((NC, B, CW), lambda i: (0, i, 0))
               for _ in range(K1)]
            + [
                pl.BlockSpec(memory_space=pltpu.SMEM),
                pl.BlockSpec((1, 1, B), lambda i: (i, 0, 0)),
                pl.BlockSpec((H, H), lambda i: (0, 0)),
                pl.BlockSpec((1, H), lambda i: (0, 0)),
                pl.BlockSpec((H, H), lambda i: (0, 0)),
                pl.BlockSpec((1, H), lambda i: (0, 0)),
                pl.BlockSpec((D, OUT), lambda i: (0, 0)),
                pl.BlockSpec((H, OUT), lambda i: (0, 0)),
                pl.BlockSpec((H, OUT), lambda i: (0, 0)),
                pl.BlockSpec((1, OUT), lambda i: (0, 0)),
            ]
        ),
        out_specs=pl.BlockSpec((G, OUT), lambda i: (0, 0)),
        out_shape=jax.ShapeDtypeStruct((G, OUT), jnp.float32),
    )(x, *h1c, *p1, s_eps, gids3, w1, b1, w2, b2, l0w, l1w, l2w, bsum)


def _fold_bn(dense, bn):
    a = bn["gamma"] / jnp.sqrt(bn["var"] + 1e-3)
    c = bn["beta"] - bn["mean"] * a
    return dense["W"] * a[None, :], (dense["b"] * a + c)[None, :]


def kernel(x, edge_index, edge_weight, graph_ids, params):
    pad = EPAD - E
    ipad = jnp.arange(pad, dtype=jnp.int32) % N
    src_r = jnp.concatenate([edge_index[0], ipad]).reshape(NW, NB, EB)
    dst_r = jnp.concatenate([edge_index[1], ipad]).reshape(NW, NB, EB)
    wbits = jax.lax.bitcast_convert_type(
        jnp.concatenate([edge_weight, jnp.zeros((pad,), jnp.float32)]),
        jnp.int32).reshape(NW, NB, EB)
    epk = jnp.stack([src_r, dst_r, wbits], axis=2)
    zeros_slab = jnp.zeros((SLAB, CW), jnp.float32)
    gids3 = graph_ids.reshape(N // 1000, 1, 1000)

    lp0, lp1 = params["layers"][0], params["layers"][1]
    w1a, b1a = _fold_bn(lp0["d1"], lp0["bn1"])
    w2a, b2a = _fold_bn(lp0["d2"], lp0["bn2"])
    w1b, b1b = _fold_bn(lp1["d1"], lp1["bn1"])
    w2b, b2b = _fold_bn(lp1["d2"], lp1["bn2"])
    lin = params["linears"]
    bsum = (lin[0]["b"] + lin[1]["b"] + lin[2]["b"])[None, :]
    s0 = (1.0 + params["eps"][0]).reshape(1, 1)
    s1 = (1.0 + params["eps"][1]).reshape(1, 1)

    x_chunks = [x[:, k * CW:(k + 1) * CW] for k in range(K0)] if K0 > 1 else [x]
    p0 = _sc_agg(x_chunks, epk, zeros_slab)
    h1c = _mlp_layer(x, p0, s0, w1a, b1a, w2a, b2a)
    p1 = _sc_agg(list(h1c), epk, zeros_slab)
    return _final_layer(x, list(h1c), list(p1), s1, gids3, w1b, b1b, w2b, b2b,
                        lin[0]["W"], lin[1]["W"], lin[2]["W"], bsum)
